# Initial kernel scaffold; baseline (speedup 1.0000x reference)
import jax
import jax.numpy as jnp
from jax.experimental import pallas as pl


def kernel(z, codebook):
    raise NotImplementedError("write your pallas kernel here")


# speedup vs baseline: 9.1108x; 9.1108x over previous
import functools

import jax
import jax.numpy as jnp
from jax import lax
from jax.experimental import pallas as pl
from jax.experimental.pallas import tpu as pltpu
from jax.experimental.pallas import tpu_sc as plsc

B = 4096
K = 1024
D = 64
BETA = 0.25

BLK = 512
NBLK = B // BLK

NC, NS = 2, 16
NW = NC * NS
BPW = B // NW


def _tc_body(z_ref, cb_ref, q_ref, idx_ref, loss_ref, loss_acc):
    i = pl.program_id(0)
    zb = z_ref[...]
    cb = cb_ref[...]
    dot = lax.dot_general(zb, cb, (((1,), (1,)), ((), ())),
                          preferred_element_type=jnp.float32)
    z2 = jnp.sum(zb * zb, axis=1, keepdims=True)
    c2 = jnp.sum(cb * cb, axis=1)
    d = z2 + c2[None, :] - 2.0 * dot

    qun = 1.0 / (1.0 + d)
    q_ref[...] = qun / jnp.sum(qun, axis=1, keepdims=True)

    iota = lax.broadcasted_iota(jnp.int32, (BLK, K), 1)
    m = jnp.max(d, axis=1, keepdims=True)
    idx = jnp.min(jnp.where(d == m, iota, K), axis=1).astype(jnp.int32)
    idx_ref[...] = idx.reshape(1, 1, BLK)

    onehot = (iota == idx[:, None]).astype(jnp.float32)
    zq = lax.dot_general(onehot, cb, (((1,), (0,)), ((), ())),
                         preferred_element_type=jnp.float32)
    r = zq - zb
    part = jnp.sum(r * r)

    @pl.when(i == 0)
    def _init():
        loss_acc[0] = 0.0

    loss_acc[0] += part

    @pl.when(i == NBLK - 1)
    def _fin():
        loss_ref[...] = (loss_acc[0] * ((1.0 + BETA) / (B * D))).reshape(1, 1)


_tc_call = pl.pallas_call(
    _tc_body,
    grid=(NBLK,),
    in_specs=[
        pl.BlockSpec((BLK, D), lambda i: (i, 0)),
        pl.BlockSpec((K, D), lambda i: (0, 0)),
    ],
    out_specs=[
        pl.BlockSpec((BLK, K), lambda i: (i, 0)),
        pl.BlockSpec((1, 1, BLK), lambda i: (i, 0, 0)),
        pl.BlockSpec((1, 1), lambda i: (0, 0)),
    ],
    out_shape=[
        jax.ShapeDtypeStruct((B, K), jnp.float32),
        jax.ShapeDtypeStruct((NBLK, 1, BLK), jnp.int32),
        jax.ShapeDtypeStruct((1, 1), jnp.float32),
    ],
    scratch_shapes=[pltpu.SMEM((1,), jnp.float32)],
)


def _sc_gather_body(cb_hbm, idx_hbm, zq_hbm, idx_v, rows_v, sem):
    wid = lax.axis_index("s") * NC + lax.axis_index("c")
    base = wid * BPW
    pltpu.sync_copy(idx_hbm.at[pl.ds(base, BPW)], idx_v)
    pltpu.async_copy(cb_hbm.at[idx_v], rows_v, sem).wait()
    pltpu.sync_copy(rows_v, zq_hbm.at[pl.ds(base, BPW)])


@functools.lru_cache(maxsize=None)
def _sc_gather_call():
    return pl.kernel(
        _sc_gather_body,
        out_type=jax.ShapeDtypeStruct((B, D), jnp.float32),
        mesh=plsc.VectorSubcoreMesh(core_axis_name="c", subcore_axis_name="s",
                                    num_cores=NC, num_subcores=NS),
        scratch_types=[
            pltpu.VMEM((BPW,), jnp.int32),
            pltpu.VMEM((BPW, D), jnp.float32),
            pltpu.SemaphoreType.DMA,
        ],
        compiler_params=pltpu.CompilerParams(use_tc_tiling_on_sc=False),
    )


def kernel(z, codebook):
    q, idx3, loss11 = _tc_call(z, codebook)
    idx = idx3.reshape(B)
    z_q = _sc_gather_call()(codebook, idx)
    loss = loss11.reshape(())
    return (q, z_q, loss)
